# baseline (speedup 1.0000x reference)
import functools

import jax
import jax.numpy as jnp
from jax.experimental import pallas as pl
from jax.experimental.pallas import tpu as pltpu

import numpy as np

_BV = 8192
_INT_MAX = np.int32(2147483647)
_TINY = np.float32(1.1754943508222875e-38)


def _threefry_gumbel(flat_idx):
    k0 = jnp.uint32(0)
    k1 = jnp.uint32(42)
    ks2 = jnp.uint32(0 ^ 42 ^ 0x1BD11BDA)
    ks = (k0, k1, ks2)
    rots = ((13, 15, 26, 6), (17, 29, 16, 24))

    x1 = flat_idx.astype(jnp.uint32) + k1
    x0 = jnp.zeros_like(x1) + k0

    def rotl(x, r):
        return (x << jnp.uint32(r)) | (x >> jnp.uint32(32 - r))

    for i in range(5):
        for r in rots[i % 2]:
            x0 = x0 + x1
            x1 = rotl(x1, r)
            x1 = x1 ^ x0
        x0 = x0 + ks[(i + 1) % 3]
        x1 = x1 + ks[(i + 2) % 3] + jnp.uint32(i + 1)

    bits = x0 ^ x1
    float_bits = (bits >> jnp.uint32(9)) | jnp.uint32(0x3F800000)
    floats = jax.lax.bitcast_convert_type(float_bits, jnp.float32) - jnp.float32(1.0)
    u = jnp.maximum(_TINY, floats * (jnp.float32(1.0) - _TINY) + _TINY)
    return -jnp.log(-jnp.log(u))


def _body(safe_t_ref, x_ref, greedy_ref, sample_ref,
          gmax, gidx, m_ref, z_ref,
          t1p, t1i, t1s, t1g, t2p, t2i, t2s, t2g,
          *, rows, vocab, nblocks):
    j = pl.program_id(0)
    shape1 = (rows, 1)

    @pl.when(j == 0)
    def _init():
        neg = jnp.full(shape1, -jnp.inf, jnp.float32)
        gmax[...] = neg
        gidx[...] = jnp.zeros(shape1, jnp.int32)
        m_ref[...] = neg
        z_ref[...] = jnp.zeros(shape1, jnp.float32)
        t1p[...] = neg
        t2p[...] = neg
        t1i[...] = jnp.full(shape1, _INT_MAX, jnp.int32)
        t2i[...] = jnp.full(shape1, _INT_MAX, jnp.int32)
        t1s[...] = jnp.zeros(shape1, jnp.float32)
        t2s[...] = jnp.zeros(shape1, jnp.float32)
        t1g[...] = jnp.zeros(shape1, jnp.float32)
        t2g[...] = jnp.zeros(shape1, jnp.float32)

    x = x_ref[...]
    col = jax.lax.broadcasted_iota(jnp.int32, x.shape, 1) + j * _BV
    valid = col < vocab
    x = jnp.where(valid, x, -jnp.inf)

    bmax = jnp.max(x, axis=1, keepdims=True)
    bidx = jnp.min(jnp.where(x == bmax, col, _INT_MAX), axis=1, keepdims=True)
    better = bmax > gmax[...]
    equal = bmax == gmax[...]
    gidx[...] = jnp.where(better, bidx,
                          jnp.where(equal, jnp.minimum(bidx, gidx[...]), gidx[...]))
    gmax[...] = jnp.maximum(bmax, gmax[...])

    s = x / safe_t_ref[...]
    row = jax.lax.broadcasted_iota(jnp.int32, x.shape, 0)
    g = _threefry_gumbel(row * vocab + col)

    m_old = m_ref[...]
    m_new = jnp.maximum(m_old, jnp.max(s, axis=1, keepdims=True))
    q = jnp.exp(s - m_new)
    z_ref[...] = z_ref[...] * jnp.exp(m_old - m_new) + jnp.sum(q, axis=1, keepdims=True)
    m_ref[...] = m_new

    shift = m_old - m_new
    p1 = t1p[...] + shift
    p2 = t2p[...] + shift

    p = (s - m_new) + g
    b1p = jnp.max(p, axis=1, keepdims=True)
    is1 = col == jnp.min(jnp.where(p == b1p, col, _INT_MAX), axis=1, keepdims=True)
    b1i = jnp.min(jnp.where(is1, col, _INT_MAX), axis=1, keepdims=True)
    b1s = jnp.max(jnp.where(is1, s, -jnp.inf), axis=1, keepdims=True)
    b1g = jnp.max(jnp.where(is1, g, -jnp.inf), axis=1, keepdims=True)

    pm = jnp.where(is1, -jnp.inf, p)
    b2p = jnp.max(pm, axis=1, keepdims=True)
    is2 = col == jnp.min(jnp.where(pm == b2p, col, _INT_MAX), axis=1, keepdims=True)
    b2i = jnp.min(jnp.where(is2, col, _INT_MAX), axis=1, keepdims=True)
    b2s = jnp.max(jnp.where(is2, s, -jnp.inf), axis=1, keepdims=True)
    b2g = jnp.max(jnp.where(is2, g, -jnp.inf), axis=1, keepdims=True)

    b1_wins = b1p > p1
    n1p = jnp.where(b1_wins, b1p, p1)
    n1i = jnp.where(b1_wins, b1i, t1i[...])
    n1s = jnp.where(b1_wins, b1s, t1s[...])
    n1g = jnp.where(b1_wins, b1g, t1g[...])
    ca_p = jnp.where(b1_wins, p1, p2)
    ca_i = jnp.where(b1_wins, t1i[...], t2i[...])
    ca_s = jnp.where(b1_wins, t1s[...], t2s[...])
    ca_g = jnp.where(b1_wins, t1g[...], t2g[...])
    cb_p = jnp.where(b1_wins, b2p, b1p)
    cb_i = jnp.where(b1_wins, b2i, b1i)
    cb_s = jnp.where(b1_wins, b2s, b1s)
    cb_g = jnp.where(b1_wins, b2g, b1g)
    b_wins2 = cb_p > ca_p
    t1p[...] = n1p
    t1i[...] = n1i
    t1s[...] = n1s
    t1g[...] = n1g
    t2p[...] = jnp.where(b_wins2, cb_p, ca_p)
    t2i[...] = jnp.where(b_wins2, cb_i, ca_i)
    t2s[...] = jnp.where(b_wins2, cb_s, ca_s)
    t2g[...] = jnp.where(b_wins2, cb_g, ca_g)

    @pl.when(j == nblocks - 1)
    def _fin():
        m = m_ref[...]
        z = z_ref[...]
        f1 = t1g[...] + jnp.log(jnp.exp(t1s[...] - m) / z + jnp.float32(1e-20))
        f2 = t2g[...] + jnp.log(jnp.exp(t2s[...] - m) / z + jnp.float32(1e-20))
        i1 = t1i[...]
        i2 = t2i[...]
        sample_ref[...] = jnp.where(
            f1 > f2, i1, jnp.where(f2 > f1, i2, jnp.minimum(i1, i2)))
        greedy_ref[...] = gidx[...]


def kernel(logits, temperatures):
    rows, vocab = logits.shape
    nblocks = (vocab + _BV - 1) // _BV
    safe_t = jnp.where(temperatures == 0, jnp.ones_like(temperatures),
                       temperatures).astype(jnp.float32).reshape(rows, 1)

    body = functools.partial(_body, rows=rows, vocab=vocab, nblocks=nblocks)

    greedy, sample = pl.pallas_call(
        body,
        grid=(nblocks,),
        in_specs=[
            pl.BlockSpec((rows, 1), lambda j: (0, 0)),
            pl.BlockSpec((rows, _BV), lambda j: (0, j)),
        ],
        out_specs=[
            pl.BlockSpec((rows, 1), lambda j: (0, 0)),
            pl.BlockSpec((rows, 1), lambda j: (0, 0)),
        ],
        out_shape=[
            jax.ShapeDtypeStruct((rows, 1), jnp.int32),
            jax.ShapeDtypeStruct((rows, 1), jnp.int32),
        ],
        scratch_shapes=[pltpu.VMEM((rows, 1), jnp.float32),
                        pltpu.VMEM((rows, 1), jnp.int32),
                        pltpu.VMEM((rows, 1), jnp.float32),
                        pltpu.VMEM((rows, 1), jnp.float32),
                        pltpu.VMEM((rows, 1), jnp.float32),
                        pltpu.VMEM((rows, 1), jnp.int32),
                        pltpu.VMEM((rows, 1), jnp.float32),
                        pltpu.VMEM((rows, 1), jnp.float32),
                        pltpu.VMEM((rows, 1), jnp.float32),
                        pltpu.VMEM((rows, 1), jnp.int32),
                        pltpu.VMEM((rows, 1), jnp.float32),
                        pltpu.VMEM((rows, 1), jnp.float32)],
        compiler_params=pltpu.CompilerParams(
            dimension_semantics=("arbitrary",)),
    )(safe_t, logits)

    greedy = greedy[:, 0]
    sample = sample[:, 0]
    return jnp.where(temperatures == 0, greedy, sample)


# speedup vs baseline: 1.1297x; 1.1297x over previous
import functools

import jax
import jax.numpy as jnp
import numpy as np
from jax.experimental import pallas as pl
from jax.experimental.pallas import tpu as pltpu

_BVM = 32768
_BVS = 8192
_INT_MAX = np.int32(2147483647)
_TINY = np.float32(1.1754943508222875e-38)


def _threefry_gumbel(x1_init):
    ks = (0, 42, 0 ^ 42 ^ 0x1BD11BDA)
    rots = ((13, 15, 26, 6), (17, 29, 16, 24))

    def rotl(x, r):
        return (x << np.uint32(r)) | (x >> np.uint32(32 - r))

    x1 = x1_init
    x0 = x1
    x1 = rotl(x1, 13) ^ x0
    for r in rots[0][1:]:
        x0 = x0 + x1
        x1 = rotl(x1, r)
        x1 = x1 ^ x0
    x0 = x0 + np.uint32(ks[1])
    x1 = x1 + np.uint32((ks[2] + 1) & 0xFFFFFFFF)

    for i in range(1, 5):
        for r in rots[i % 2]:
            x0 = x0 + x1
            x1 = rotl(x1, r)
            x1 = x1 ^ x0
        if ks[(i + 1) % 3]:
            x0 = x0 + np.uint32(ks[(i + 1) % 3])
        x1 = x1 + np.uint32((ks[(i + 2) % 3] + i + 1) & 0xFFFFFFFF)

    bits = x0 ^ x1
    float_bits = (bits >> jnp.uint32(9)) | jnp.uint32(0x3F800000)
    floats = jax.lax.bitcast_convert_type(float_bits, jnp.float32) - jnp.float32(1.0)
    u = jnp.maximum(floats, _TINY)
    return -jnp.log(-jnp.log(u))


def _max_body(safe_t_ref, x_ref, m_ref, acc, *, vocab, nblocks):
    j = pl.program_id(0)

    @pl.when(j == 0)
    def _init():
        acc[...] = jnp.full(acc.shape, -jnp.inf, jnp.float32)

    x = x_ref[...]
    col = jax.lax.broadcasted_iota(jnp.int32, x.shape, 1) + j * _BVM
    x = jnp.where(col < vocab, x, -jnp.inf)
    s = x / safe_t_ref[...]
    acc[...] = jnp.maximum(acc[...], jnp.max(s, axis=1, keepdims=True))

    @pl.when(j == nblocks - 1)
    def _fin():
        m_ref[...] = acc[...]


def _sample_body(safe_t_ref, row_off_ref, m_in_ref, gscale_ref, x_ref,
                 samp_ref, best_p, best_i, *, vocab, nblocks):
    j = pl.program_id(0)

    @pl.when(j == 0)
    def _init():
        best_p[...] = jnp.full(best_p.shape, -jnp.inf, jnp.float32)
        best_i[...] = jnp.full(best_i.shape, _INT_MAX, jnp.int32)

    x = x_ref[...]
    col = jax.lax.broadcasted_iota(jnp.int32, x.shape, 1) + j * _BVS
    x = jnp.where(col < vocab, x, -jnp.inf)
    s = x / safe_t_ref[...]

    x1 = (col + row_off_ref[...]).astype(jnp.uint32)
    g = _threefry_gumbel(x1) * gscale_ref[...]

    p = (s - m_in_ref[...]) + g
    bp = jnp.max(p, axis=1, keepdims=True)
    bi = jnp.min(jnp.where(p == bp, col, _INT_MAX), axis=1, keepdims=True)
    upd = bp > best_p[...]
    best_i[...] = jnp.where(upd, bi, best_i[...])
    best_p[...] = jnp.where(upd, bp, best_p[...])

    @pl.when(j == nblocks - 1)
    def _fin():
        samp_ref[...] = best_i[...]


def kernel(logits, temperatures):
    rows, vocab = logits.shape
    nbm = (vocab + _BVM - 1) // _BVM
    nbs = (vocab + _BVS - 1) // _BVS

    tcol = temperatures.astype(jnp.float32).reshape(rows, 1)
    zero_t = tcol == 0
    safe_t = jnp.where(zero_t, jnp.ones_like(tcol), tcol)
    gscale = jnp.where(zero_t, jnp.zeros_like(tcol), jnp.ones_like(tcol))
    row_off = (jnp.arange(rows, dtype=jnp.int32) * vocab + 42).reshape(rows, 1)

    small = pl.BlockSpec((rows, 1), lambda j: (0, 0))

    m = pl.pallas_call(
        functools.partial(_max_body, vocab=vocab, nblocks=nbm),
        grid=(nbm,),
        in_specs=[small, pl.BlockSpec((rows, _BVM), lambda j: (0, j))],
        out_specs=small,
        out_shape=jax.ShapeDtypeStruct((rows, 1), jnp.float32),
        scratch_shapes=[pltpu.VMEM((rows, 1), jnp.float32)],
        compiler_params=pltpu.CompilerParams(
            dimension_semantics=("arbitrary",)),
    )(safe_t, logits)

    samp = pl.pallas_call(
        functools.partial(_sample_body, vocab=vocab, nblocks=nbs),
        grid=(nbs,),
        in_specs=[small, small, small, small,
                  pl.BlockSpec((rows, _BVS), lambda j: (0, j))],
        out_specs=small,
        out_shape=jax.ShapeDtypeStruct((rows, 1), jnp.int32),
        scratch_shapes=[pltpu.VMEM((rows, 1), jnp.float32),
                        pltpu.VMEM((rows, 1), jnp.int32)],
        compiler_params=pltpu.CompilerParams(
            dimension_semantics=("arbitrary",)),
    )(safe_t, row_off, m, gscale, logits)

    return samp[:, 0]


# speedup vs baseline: 1.5443x; 1.3671x over previous
import functools

import jax
import jax.numpy as jnp
import numpy as np
from jax.experimental import pallas as pl
from jax.experimental.pallas import tpu as pltpu

_BVM = 32768
_BVS = 8192
_BVC = 2048
_INT_MAX = np.int32(2147483647)
_TINY = np.float32(1.1754943508222875e-38)


def _threefry_gumbel(x1_init):
    ks = (0, 42, 0 ^ 42 ^ 0x1BD11BDA)
    rots = ((13, 15, 26, 6), (17, 29, 16, 24))

    def rotl(x, r):
        return (x << np.uint32(r)) | (x >> np.uint32(32 - r))

    x1 = x1_init
    x0 = x1
    x1 = rotl(x1, 13) ^ x0
    for r in rots[0][1:]:
        x0 = x0 + x1
        x1 = rotl(x1, r)
        x1 = x1 ^ x0
    x0 = x0 + np.uint32(ks[1])
    x1 = x1 + np.uint32((ks[2] + 1) & 0xFFFFFFFF)

    for i in range(1, 5):
        for r in rots[i % 2]:
            x0 = x0 + x1
            x1 = rotl(x1, r)
            x1 = x1 ^ x0
        if ks[(i + 1) % 3]:
            x0 = x0 + np.uint32(ks[(i + 1) % 3])
        x1 = x1 + np.uint32((ks[(i + 2) % 3] + i + 1) & 0xFFFFFFFF)

    bits = x0 ^ x1
    float_bits = (bits >> jnp.uint32(9)) | jnp.uint32(0x3F800000)
    floats = jax.lax.bitcast_convert_type(float_bits, jnp.float32) - jnp.float32(1.0)
    u = jnp.maximum(floats, _TINY)
    return -jnp.log(-jnp.log(u))


def _max_body(safe_t_ref, x_ref, m_ref, acc, *, vocab, nblocks):
    j = pl.program_id(0)

    @pl.when(j == 0)
    def _init():
        acc[...] = jnp.full(acc.shape, -jnp.inf, jnp.float32)

    x = x_ref[...]
    col = jax.lax.broadcasted_iota(jnp.int32, x.shape, 1) + j * _BVM
    x = jnp.where(col < vocab, x, -jnp.inf)
    s = x / safe_t_ref[...]
    acc[...] = jnp.maximum(acc[...], jnp.max(s, axis=1, keepdims=True))

    @pl.when(j == nblocks - 1)
    def _fin():
        m_ref[...] = acc[...]


def _sample_body(safe_t_ref, row_off_ref, m_in_ref, gscale_ref, x_ref,
                 samp_ref, best_p, best_i, *, vocab, nblocks):
    j = pl.program_id(0)

    @pl.when(j == 0)
    def _init():
        best_p[...] = jnp.full(best_p.shape, -jnp.inf, jnp.float32)
        best_i[...] = jnp.full(best_i.shape, _INT_MAX, jnp.int32)

    for c in range(_BVS // _BVC):
        x = x_ref[:, c * _BVC:(c + 1) * _BVC]
        col = (jax.lax.broadcasted_iota(jnp.int32, x.shape, 1)
               + (j * _BVS + c * _BVC))
        x = jnp.where(col < vocab, x, -jnp.inf)
        s = x / safe_t_ref[...]

        x1 = (col + row_off_ref[...]).astype(jnp.uint32)
        g = _threefry_gumbel(x1) * gscale_ref[...]

        p = (s - m_in_ref[...]) + g
        bp = jnp.max(p, axis=1, keepdims=True)
        bi = jnp.min(jnp.where(p == bp, col, _INT_MAX), axis=1, keepdims=True)
        upd = bp > best_p[...]
        best_i[...] = jnp.where(upd, bi, best_i[...])
        best_p[...] = jnp.where(upd, bp, best_p[...])

    @pl.when(j == nblocks - 1)
    def _fin():
        samp_ref[...] = best_i[...]


def kernel(logits, temperatures):
    rows, vocab = logits.shape
    nbm = (vocab + _BVM - 1) // _BVM
    nbs = (vocab + _BVS - 1) // _BVS

    tcol = temperatures.astype(jnp.float32).reshape(rows, 1)
    zero_t = tcol == 0
    safe_t = jnp.where(zero_t, jnp.ones_like(tcol), tcol)
    gscale = jnp.where(zero_t, jnp.zeros_like(tcol), jnp.ones_like(tcol))
    row_off = (jnp.arange(rows, dtype=jnp.int32) * vocab + 42).reshape(rows, 1)

    small = pl.BlockSpec((rows, 1), lambda j: (0, 0))

    m = pl.pallas_call(
        functools.partial(_max_body, vocab=vocab, nblocks=nbm),
        grid=(nbm,),
        in_specs=[small, pl.BlockSpec((rows, _BVM), lambda j: (0, j))],
        out_specs=small,
        out_shape=jax.ShapeDtypeStruct((rows, 1), jnp.float32),
        scratch_shapes=[pltpu.VMEM((rows, 1), jnp.float32)],
        compiler_params=pltpu.CompilerParams(
            dimension_semantics=("arbitrary",)),
    )(safe_t, logits)

    samp = pl.pallas_call(
        functools.partial(_sample_body, vocab=vocab, nblocks=nbs),
        grid=(nbs,),
        in_specs=[small, small, small, small,
                  pl.BlockSpec((rows, _BVS), lambda j: (0, j))],
        out_specs=small,
        out_shape=jax.ShapeDtypeStruct((rows, 1), jnp.int32),
        scratch_shapes=[pltpu.VMEM((rows, 1), jnp.float32),
                        pltpu.VMEM((rows, 1), jnp.int32)],
        compiler_params=pltpu.CompilerParams(
            dimension_semantics=("arbitrary",)),
    )(safe_t, row_off, m, gscale, logits)

    return samp[:, 0]


# speedup vs baseline: 1.5652x; 1.0135x over previous
import functools

import jax
import jax.numpy as jnp
import numpy as np
from jax.experimental import pallas as pl
from jax.experimental.pallas import tpu as pltpu

_BV = 16384
_BVC = 2048
_INT_MAX = np.int32(2147483647)
_TINY = np.float32(1.1754943508222875e-38)


def _threefry_gumbel(x1_init):
    ks = (0, 42, 0 ^ 42 ^ 0x1BD11BDA)
    rots = ((13, 15, 26, 6), (17, 29, 16, 24))

    def rotl(x, r):
        return (x << np.uint32(r)) | (x >> np.uint32(32 - r))

    x1 = x1_init
    x0 = x1
    x1 = rotl(x1, 13) ^ x0
    for r in rots[0][1:]:
        x0 = x0 + x1
        x1 = rotl(x1, r)
        x1 = x1 ^ x0
    x0 = x0 + np.uint32(ks[1])
    x1 = x1 + np.uint32((ks[2] + 1) & 0xFFFFFFFF)

    for i in range(1, 5):
        for r in rots[i % 2]:
            x0 = x0 + x1
            x1 = rotl(x1, r)
            x1 = x1 ^ x0
        if ks[(i + 1) % 3]:
            x0 = x0 + np.uint32(ks[(i + 1) % 3])
        x1 = x1 + np.uint32((ks[(i + 2) % 3] + i + 1) & 0xFFFFFFFF)

    bits = x0 ^ x1
    float_bits = (bits >> jnp.uint32(9)) | jnp.uint32(0x3F800000)
    floats = jax.lax.bitcast_convert_type(float_bits, jnp.float32) - jnp.float32(1.0)
    u = jnp.maximum(floats, _TINY)
    return -jnp.log(-jnp.log(u))


def _body(t_ref, row_off_ref, x_ref, samp_ref,
          acc, m_s, gs_s, best_p, best_i, *, vocab, nblocks):
    ph = pl.program_id(0)
    j = pl.program_id(1)
    last = nblocks - 1

    @pl.when(ph == 0)
    def _phase_max():
        @pl.when(j == 0)
        def _init():
            acc[...] = jnp.full(acc.shape, -jnp.inf, jnp.float32)

        x = x_ref[...]

        @pl.when(j != last)
        def _full():
            acc[...] = jnp.maximum(acc[...], jnp.max(x, axis=1, keepdims=True))

        @pl.when(j == last)
        def _tail():
            col = jax.lax.broadcasted_iota(jnp.int32, x.shape, 1) + j * _BV
            xm = jnp.where(col < vocab, x, -jnp.inf)
            acc[...] = jnp.maximum(acc[...], jnp.max(xm, axis=1, keepdims=True))
            t = t_ref[...]
            zt = t == 0
            safe_t = jnp.where(zt, jnp.ones_like(t), t)
            m_s[...] = acc[...] / safe_t
            gs_s[...] = jnp.where(zt, jnp.zeros_like(t), jnp.ones_like(t))
            best_p[...] = jnp.full(best_p.shape, -jnp.inf, jnp.float32)
            best_i[...] = jnp.full(best_i.shape, _INT_MAX, jnp.int32)

    @pl.when(ph == 1)
    def _phase_sample():
        t = t_ref[...]
        safe_t = jnp.where(t == 0, jnp.ones_like(t), t)
        m = m_s[...]
        gscale = gs_s[...]

        def chunk(c, masked):
            x = x_ref[:, c * _BVC:(c + 1) * _BVC]
            col = (jax.lax.broadcasted_iota(jnp.int32, x.shape, 1)
                   + (j * _BV + c * _BVC))
            if masked:
                x = jnp.where(col < vocab, x, -jnp.inf)
            s = x / safe_t
            x1 = (col + row_off_ref[...]).astype(jnp.uint32)
            g = _threefry_gumbel(x1) * gscale
            p = (s - m) + g
            bp = jnp.max(p, axis=1, keepdims=True)
            bi = jnp.min(jnp.where(p == bp, col, _INT_MAX), axis=1,
                         keepdims=True)
            upd = bp > best_p[...]
            best_i[...] = jnp.where(upd, bi, best_i[...])
            best_p[...] = jnp.where(upd, bp, best_p[...])

        @pl.when(j != last)
        def _full():
            for c in range(_BV // _BVC):
                chunk(c, masked=False)

        @pl.when(j == last)
        def _tail():
            for c in range(_BV // _BVC):
                chunk(c, masked=True)

        @pl.when(j == last)
        def _fin():
            samp_ref[...] = best_i[...]


def kernel(logits, temperatures):
    rows, vocab = logits.shape
    nblocks = (vocab + _BV - 1) // _BV

    tcol = temperatures.astype(jnp.float32).reshape(rows, 1)
    row_off = np.arange(rows, dtype=np.int32).reshape(rows, 1) * vocab + 42

    small = pl.BlockSpec((rows, 1), lambda ph, j: (0, 0))

    samp = pl.pallas_call(
        functools.partial(_body, vocab=vocab, nblocks=nblocks),
        grid=(2, nblocks),
        in_specs=[small, small,
                  pl.BlockSpec((rows, _BV), lambda ph, j: (0, j))],
        out_specs=small,
        out_shape=jax.ShapeDtypeStruct((rows, 1), jnp.int32),
        scratch_shapes=[pltpu.VMEM((rows, 1), jnp.float32),
                        pltpu.VMEM((rows, 1), jnp.float32),
                        pltpu.VMEM((rows, 1), jnp.float32),
                        pltpu.VMEM((rows, 1), jnp.float32),
                        pltpu.VMEM((rows, 1), jnp.int32)],
        compiler_params=pltpu.CompilerParams(
            dimension_semantics=("arbitrary", "arbitrary")),
    )(tcol, jnp.asarray(row_off), logits)

    return samp[:, 0]


# speedup vs baseline: 1.6517x; 1.0552x over previous
import functools

import jax
import jax.numpy as jnp
import numpy as np
from jax.experimental import pallas as pl
from jax.experimental.pallas import tpu as pltpu

_BV = 16384
_BVC = 2048
_INT_MAX = np.int32(2147483647)
_TINY = np.float32(1.1754943508222875e-38)


def _threefry_gumbel(x1_init):
    ks = (0, 42, 0 ^ 42 ^ 0x1BD11BDA)
    rots = ((13, 15, 26, 6), (17, 29, 16, 24))

    def rotl(x, r):
        return (x << np.uint32(r)) | (x >> np.uint32(32 - r))

    x1 = x1_init
    x0 = x1
    x1 = rotl(x1, 13) ^ x0
    for r in rots[0][1:]:
        x0 = x0 + x1
        x1 = rotl(x1, r)
        x1 = x1 ^ x0
    x0 = x0 + np.uint32(ks[1])
    x1 = x1 + np.uint32((ks[2] + 1) & 0xFFFFFFFF)

    for i in range(1, 5):
        for r in rots[i % 2]:
            x0 = x0 + x1
            x1 = rotl(x1, r)
            x1 = x1 ^ x0
        if ks[(i + 1) % 3]:
            x0 = x0 + np.uint32(ks[(i + 1) % 3])
        x1 = x1 + np.uint32((ks[(i + 2) % 3] + i + 1) & 0xFFFFFFFF)

    bits = x0 ^ x1
    float_bits = (bits >> jnp.uint32(9)) | jnp.uint32(0x3F800000)
    floats = jax.lax.bitcast_convert_type(float_bits, jnp.float32) - jnp.float32(1.0)
    u = jnp.maximum(floats, _TINY)
    return -jnp.log(-jnp.log(u))


def _body(t_ref, row_off_ref, xmax_ref, xsmp_ref, samp_ref,
          m_prev, mcol, pcol, icol, *, rows, vocab, nblocks, ncol):
    j = pl.program_id(0)

    @pl.when(j == 0)
    def _init():
        mcol[...] = jnp.full(mcol.shape, -jnp.inf, jnp.float32)
        pcol[...] = jnp.full(pcol.shape, -jnp.inf, jnp.float32)
        icol[...] = jnp.zeros(icol.shape, jnp.int32)

    @pl.when(j > 0)
    def _sample():
        t = t_ref[...]
        safe_t = jnp.where(t == 0, jnp.ones_like(t), t)
        gscale = jnp.where(t == 0, jnp.zeros_like(t), jnp.ones_like(t))
        m = m_prev[...] / safe_t
        base = (j - 1) * _BV

        def run(masked):
            bp = jnp.full((rows, 1), -jnp.inf, jnp.float32)
            bi = jnp.full((rows, 1), _INT_MAX, jnp.int32)
            for c in range(_BV // _BVC):
                x = xsmp_ref[:, c * _BVC:(c + 1) * _BVC]
                col = (jax.lax.broadcasted_iota(jnp.int32, x.shape, 1)
                       + (base + c * _BVC))
                if masked:
                    x = jnp.where(col < vocab, x, -jnp.inf)
                s = x / safe_t
                x1 = (col + row_off_ref[...]).astype(jnp.uint32)
                g = _threefry_gumbel(x1) * gscale
                p = (s - m) + g
                cp = jnp.max(p, axis=1, keepdims=True)
                ci = jnp.min(jnp.where(p == cp, col, _INT_MAX), axis=1,
                             keepdims=True)
                upd = cp > bp
                bi = jnp.where(upd, ci, bi)
                bp = jnp.where(upd, cp, bp)
            pcol[pl.ds(j - 1, 1)] = bp[None]
            icol[pl.ds(j - 1, 1)] = bi[None]
            mcol[pl.ds(j - 1, 1)] = m_prev[...][None]

        @pl.when(j != nblocks)
        def _full():
            run(masked=False)

        @pl.when(j == nblocks)
        def _tail():
            run(masked=True)

    @pl.when(j < nblocks)
    def _maxblock():
        x = xmax_ref[...]

        @pl.when(j != nblocks - 1)
        def _full():
            m_prev[...] = jnp.max(x, axis=1, keepdims=True)

        @pl.when(j == nblocks - 1)
        def _tail():
            col = jax.lax.broadcasted_iota(jnp.int32, x.shape, 1) + j * _BV
            xm = jnp.where(col < vocab, x, -jnp.inf)
            m_prev[...] = jnp.max(xm, axis=1, keepdims=True)

    @pl.when(j == nblocks)
    def _merge():
        t = t_ref[...]
        safe_t = jnp.where(t == 0, jnp.ones_like(t), t)
        ms = mcol[...] / safe_t
        gm = jnp.max(ms, axis=0, keepdims=True)
        shifted = pcol[...] + (ms - gm)
        best = jnp.max(shifted, axis=0, keepdims=True)
        blk_iota = jax.lax.broadcasted_iota(jnp.int32, shifted.shape, 0)
        blk = jnp.min(jnp.where(shifted == best, blk_iota, _INT_MAX), axis=0,
                      keepdims=True)
        samp_ref[...] = jnp.min(
            jnp.where(blk_iota == blk, icol[...], _INT_MAX), axis=0)


def kernel(logits, temperatures):
    rows, vocab = logits.shape
    nblocks = (vocab + _BV - 1) // _BV
    ncol = nblocks

    tcol = temperatures.astype(jnp.float32).reshape(rows, 1)
    row_off = np.arange(rows, dtype=np.int32).reshape(rows, 1) * vocab + 42

    small = pl.BlockSpec((rows, 1), lambda j: (0, 0))
    last = nblocks - 1

    samp = pl.pallas_call(
        functools.partial(_body, rows=rows, vocab=vocab, nblocks=nblocks,
                          ncol=ncol),
        grid=(nblocks + 1,),
        in_specs=[small, small,
                  pl.BlockSpec((rows, _BV),
                               lambda j: (0, jnp.minimum(j, last))),
                  pl.BlockSpec((rows, _BV),
                               lambda j: (0, jnp.maximum(j - 1, 0)))],
        out_specs=small,
        out_shape=jax.ShapeDtypeStruct((rows, 1), jnp.int32),
        scratch_shapes=[pltpu.VMEM((rows, 1), jnp.float32),
                        pltpu.VMEM((ncol, rows, 1), jnp.float32),
                        pltpu.VMEM((ncol, rows, 1), jnp.float32),
                        pltpu.VMEM((ncol, rows, 1), jnp.int32)],
        compiler_params=pltpu.CompilerParams(
            dimension_semantics=("arbitrary",)),
    )(tcol, jnp.asarray(row_off), logits, logits)

    return samp[:, 0]
